# baseline (device time: 67347 ns/iter reference)
import jax
import jax.numpy as jnp
from jax import lax
from jax.experimental import pallas as pl
from jax.experimental.pallas import tpu as pltpu

R = 512
LANES = 128
USE_ROLL = True
OUT_DTYPE = jnp.bfloat16


def _exchange(row_edge, col_packed, x):
    m, n = x.shape
    nb = m // R
    pk = m // LANES

    def body(row_ref, col_ref, x_ref, up_ref, dn_ref, ocol_ref,
             rsend, rrecv, lsem):
        my_x = lax.axis_index("x")
        my_y = lax.axis_index("y")

        for i in range(1, nb):
            pltpu.make_async_copy(
                x_ref.at[pl.ds(i * R - 1, 1), :],
                up_ref.at[pl.ds(i, 1), :],
                lsem.at[0, i],
            ).start()
        for i in range(nb - 1):
            pltpu.make_async_copy(
                x_ref.at[pl.ds((i + 1) * R, 1), :],
                dn_ref.at[pl.ds(i, 1), :],
                lsem.at[1, i],
            ).start()

        bsem = pltpu.get_barrier_semaphore()
        for nbr in ((1 - my_x, my_y), (my_x, 1 - my_y)):
            pl.semaphore_signal(
                bsem, inc=1, device_id=nbr,
                device_id_type=pl.DeviceIdType.MESH,
            )
        pl.semaphore_wait(bsem, 2)

        @pl.when(my_x == 0)
        def _():
            rdma = pltpu.make_async_remote_copy(
                src_ref=row_ref,
                dst_ref=up_ref.at[pl.ds(0, 1), :],
                send_sem=rsend.at[0],
                recv_sem=rrecv.at[0],
                device_id=(1 - my_x, my_y),
                device_id_type=pl.DeviceIdType.MESH,
            )
            rdma.start()
            rdma.wait()

        @pl.when(my_x == 1)
        def _():
            rdma = pltpu.make_async_remote_copy(
                src_ref=row_ref,
                dst_ref=dn_ref.at[pl.ds(nb - 1, 1), :],
                send_sem=rsend.at[0],
                recv_sem=rrecv.at[0],
                device_id=(1 - my_x, my_y),
                device_id_type=pl.DeviceIdType.MESH,
            )
            rdma.start()
            rdma.wait()

        c_rdma = pltpu.make_async_remote_copy(
            src_ref=col_ref,
            dst_ref=ocol_ref,
            send_sem=rsend.at[1],
            recv_sem=rrecv.at[1],
            device_id=(my_x, 1 - my_y),
            device_id_type=pl.DeviceIdType.MESH,
        )
        c_rdma.start()
        c_rdma.wait()

        for i in range(1, nb):
            pltpu.make_async_copy(
                x_ref.at[pl.ds(i * R - 1, 1), :],
                up_ref.at[pl.ds(i, 1), :],
                lsem.at[0, i],
            ).wait()
        for i in range(nb - 1):
            pltpu.make_async_copy(
                x_ref.at[pl.ds((i + 1) * R, 1), :],
                dn_ref.at[pl.ds(i, 1), :],
                lsem.at[1, i],
            ).wait()

    return pl.pallas_call(
        body,
        out_shape=[
            jax.ShapeDtypeStruct((nb, n), x.dtype),
            jax.ShapeDtypeStruct((nb, n), x.dtype),
            jax.ShapeDtypeStruct((pk, LANES), x.dtype),
        ],
        in_specs=[
            pl.BlockSpec(memory_space=pltpu.VMEM),
            pl.BlockSpec(memory_space=pltpu.VMEM),
            pl.BlockSpec(memory_space=pltpu.MemorySpace.HBM),
        ],
        out_specs=[
            pl.BlockSpec(memory_space=pltpu.VMEM),
            pl.BlockSpec(memory_space=pltpu.VMEM),
            pl.BlockSpec(memory_space=pltpu.VMEM),
        ],
        scratch_shapes=[
            pltpu.SemaphoreType.DMA((2,)),
            pltpu.SemaphoreType.DMA((2,)),
            pltpu.SemaphoreType.DMA((2, nb)),
        ],
        compiler_params=pltpu.CompilerParams(collective_id=0),
    )(row_edge, col_packed, x)


def _stencil(x, up_edges, dn_edges, recv_col):
    m, n = x.shape
    nb = m // R

    def body(x_ref, up_ref, dn_ref, wcol_ref, ecol_ref, out_ref):
        i = pl.program_id(0)
        my_x = lax.axis_index("x")
        my_y = lax.axis_index("y")

        xb = x_ref[...]
        up_row = up_ref[pl.ds(i, 1), :]
        dn_row = dn_ref[pl.ds(i, 1), :]
        up = jnp.concatenate([up_row, xb[:-1, :]], axis=0)
        dn = jnp.concatenate([xb[1:, :], dn_row], axis=0)
        if USE_ROLL:
            left = pltpu.roll(xb, 1, 1)
            right = pltpu.roll(xb, n - 1, 1)
        else:
            left = jnp.concatenate([wcol_ref[...], xb[:, :-1]], axis=1)
            right = jnp.concatenate([xb[:, 1:], ecol_ref[...]], axis=1)
        st = 0.5 * xb + 0.125 * (up + dn + left + right)
        out_ref[...] = st.astype(OUT_DTYPE)

        if USE_ROLL:
            lfix = 0.5 * xb[:, 0:1] + 0.125 * (
                up[:, 0:1] + dn[:, 0:1] + wcol_ref[...] + xb[:, 1:2]
            )
            out_ref[:, 0:1] = lfix.astype(OUT_DTYPE)
            rfix = 0.5 * xb[:, n - 1 : n] + 0.125 * (
                up[:, n - 1 : n]
                + dn[:, n - 1 : n]
                + xb[:, n - 2 : n - 1]
                + ecol_ref[...]
            )
            out_ref[:, n - 1 : n] = rfix.astype(OUT_DTYPE)

        @pl.when(my_y == 0)
        def _():
            out_ref[:, 0:1] = xb[:, 0:1].astype(OUT_DTYPE)

        @pl.when(my_y == 1)
        def _():
            out_ref[:, n - 1 : n] = xb[:, n - 1 : n].astype(OUT_DTYPE)

        @pl.when((my_x == 0) & (i == 0))
        def _():
            out_ref[0:1, :] = xb[0:1, :].astype(OUT_DTYPE)

        @pl.when((my_x == 1) & (i == nb - 1))
        def _():
            out_ref[R - 1 : R, :] = xb[R - 1 : R, :].astype(OUT_DTYPE)

    return pl.pallas_call(
        body,
        grid=(nb,),
        out_shape=jax.ShapeDtypeStruct((m, n), OUT_DTYPE),
        in_specs=[
            pl.BlockSpec((R, n), lambda i: (i, 0)),
            pl.BlockSpec((nb, n), lambda i: (0, 0)),
            pl.BlockSpec((nb, n), lambda i: (0, 0)),
            pl.BlockSpec((R, 1), lambda i: (i, 0)),
            pl.BlockSpec((R, 1), lambda i: (i, 0)),
        ],
        out_specs=pl.BlockSpec((R, n), lambda i: (i, 0)),
        compiler_params=pltpu.CompilerParams(
            vmem_limit_bytes=100 * 1024 * 1024,
        ),
    )(x, up_edges, dn_edges, recv_col, recv_col)


def kernel(x):
    m, n = x.shape
    my_x = lax.axis_index("x")
    my_y = lax.axis_index("y")

    row_edge = lax.dynamic_slice(x, ((1 - my_x) * (m - 1), 0), (1, n))
    col_edge = lax.dynamic_slice(x, (0, (1 - my_y) * (n - 1)), (m, 1))
    col_packed = col_edge.reshape(m // LANES, LANES)

    up_edges, dn_edges, recv_col_packed = _exchange(row_edge, col_packed, x)
    recv_col = recv_col_packed.reshape(m, 1)

    return _stencil(x, up_edges, dn_edges, recv_col)


# device time: 57612 ns/iter; 1.1690x vs baseline; 1.1690x over previous
import jax
import jax.numpy as jnp
from jax import lax
from jax.experimental import pallas as pl
from jax.experimental.pallas import tpu as pltpu

R = 256
LANES = 128
OUT_DTYPE = jnp.bfloat16


def _exchange(row_edge, col_packed, x):
    m, n = x.shape
    nb = m // R
    pk = m // LANES

    def body(row_ref, col_ref, x_ref, up_ref, dn_ref, ocol_ref,
             rsend, rrecv, lsem):
        my_x = lax.axis_index("x")
        my_y = lax.axis_index("y")

        for i in range(1, nb):
            pltpu.make_async_copy(
                x_ref.at[pl.ds(i * R - 1, 1), :],
                up_ref.at[pl.ds(i, 1), :],
                lsem.at[0, i],
            ).start()
        for i in range(nb - 1):
            pltpu.make_async_copy(
                x_ref.at[pl.ds((i + 1) * R, 1), :],
                dn_ref.at[pl.ds(i, 1), :],
                lsem.at[1, i],
            ).start()

        bsem = pltpu.get_barrier_semaphore()
        for nbr in ((1 - my_x, my_y), (my_x, 1 - my_y)):
            pl.semaphore_signal(
                bsem, inc=1, device_id=nbr,
                device_id_type=pl.DeviceIdType.MESH,
            )
        pl.semaphore_wait(bsem, 2)

        @pl.when(my_x == 0)
        def _():
            rdma = pltpu.make_async_remote_copy(
                src_ref=row_ref,
                dst_ref=up_ref.at[pl.ds(0, 1), :],
                send_sem=rsend.at[0],
                recv_sem=rrecv.at[0],
                device_id=(1 - my_x, my_y),
                device_id_type=pl.DeviceIdType.MESH,
            )
            rdma.start()
            rdma.wait()

        @pl.when(my_x == 1)
        def _():
            rdma = pltpu.make_async_remote_copy(
                src_ref=row_ref,
                dst_ref=dn_ref.at[pl.ds(nb - 1, 1), :],
                send_sem=rsend.at[0],
                recv_sem=rrecv.at[0],
                device_id=(1 - my_x, my_y),
                device_id_type=pl.DeviceIdType.MESH,
            )
            rdma.start()
            rdma.wait()

        c_rdma = pltpu.make_async_remote_copy(
            src_ref=col_ref,
            dst_ref=ocol_ref,
            send_sem=rsend.at[1],
            recv_sem=rrecv.at[1],
            device_id=(my_x, 1 - my_y),
            device_id_type=pl.DeviceIdType.MESH,
        )
        c_rdma.start()
        c_rdma.wait()

        for i in range(1, nb):
            pltpu.make_async_copy(
                x_ref.at[pl.ds(i * R - 1, 1), :],
                up_ref.at[pl.ds(i, 1), :],
                lsem.at[0, i],
            ).wait()
        for i in range(nb - 1):
            pltpu.make_async_copy(
                x_ref.at[pl.ds((i + 1) * R, 1), :],
                dn_ref.at[pl.ds(i, 1), :],
                lsem.at[1, i],
            ).wait()

    return pl.pallas_call(
        body,
        out_shape=[
            jax.ShapeDtypeStruct((nb, n), x.dtype),
            jax.ShapeDtypeStruct((nb, n), x.dtype),
            jax.ShapeDtypeStruct((pk, LANES), x.dtype),
        ],
        in_specs=[
            pl.BlockSpec(memory_space=pltpu.VMEM),
            pl.BlockSpec(memory_space=pltpu.VMEM),
            pl.BlockSpec(memory_space=pltpu.MemorySpace.HBM),
        ],
        out_specs=[
            pl.BlockSpec(memory_space=pltpu.VMEM),
            pl.BlockSpec(memory_space=pltpu.VMEM),
            pl.BlockSpec(memory_space=pltpu.VMEM),
        ],
        scratch_shapes=[
            pltpu.SemaphoreType.DMA((2,)),
            pltpu.SemaphoreType.DMA((2,)),
            pltpu.SemaphoreType.DMA((2, nb)),
        ],
        compiler_params=pltpu.CompilerParams(collective_id=0),
    )(row_edge, col_packed, x)


def _stencil(x, up_edges, dn_edges, recv_col):
    m, n = x.shape
    nb = m // R

    def body(x_ref, up_ref, dn_ref, wcol_ref, ecol_ref, out_ref):
        i = pl.program_id(0)
        my_x = lax.axis_index("x")
        my_y = lax.axis_index("y")

        half = jnp.bfloat16(0.5)
        eighth = jnp.bfloat16(0.125)
        xb = x_ref[...].astype(OUT_DTYPE)
        up_row = up_ref[pl.ds(i, 1), :].astype(OUT_DTYPE)
        dn_row = dn_ref[pl.ds(i, 1), :].astype(OUT_DTYPE)
        up = jnp.concatenate([up_row, xb[:-1, :]], axis=0)
        dn = jnp.concatenate([xb[1:, :], dn_row], axis=0)
        left = pltpu.roll(xb, 1, 1)
        right = pltpu.roll(xb, n - 1, 1)
        out_ref[...] = half * xb + eighth * ((up + dn) + (left + right))

        wcol = wcol_ref[...].astype(OUT_DTYPE)
        ecol = ecol_ref[...].astype(OUT_DTYPE)
        out_ref[:, 0:1] = half * xb[:, 0:1] + eighth * (
            up[:, 0:1] + dn[:, 0:1] + wcol + xb[:, 1:2]
        )
        out_ref[:, n - 1 : n] = half * xb[:, n - 1 : n] + eighth * (
            up[:, n - 1 : n] + dn[:, n - 1 : n] + xb[:, n - 2 : n - 1] + ecol
        )

        @pl.when(my_y == 0)
        def _():
            out_ref[:, 0:1] = xb[:, 0:1]

        @pl.when(my_y == 1)
        def _():
            out_ref[:, n - 1 : n] = xb[:, n - 1 : n]

        @pl.when((my_x == 0) & (i == 0))
        def _():
            out_ref[0:1, :] = xb[0:1, :]

        @pl.when((my_x == 1) & (i == nb - 1))
        def _():
            out_ref[R - 1 : R, :] = xb[R - 1 : R, :]

    return pl.pallas_call(
        body,
        grid=(nb,),
        out_shape=jax.ShapeDtypeStruct((m, n), OUT_DTYPE),
        in_specs=[
            pl.BlockSpec((R, n), lambda i: (i, 0)),
            pl.BlockSpec((nb, n), lambda i: (0, 0)),
            pl.BlockSpec((nb, n), lambda i: (0, 0)),
            pl.BlockSpec((R, 1), lambda i: (i, 0)),
            pl.BlockSpec((R, 1), lambda i: (i, 0)),
        ],
        out_specs=pl.BlockSpec((R, n), lambda i: (i, 0)),
        compiler_params=pltpu.CompilerParams(
            vmem_limit_bytes=100 * 1024 * 1024,
        ),
    )(x, up_edges, dn_edges, recv_col, recv_col)


def kernel(x):
    m, n = x.shape
    my_x = lax.axis_index("x")
    my_y = lax.axis_index("y")

    row_edge = lax.dynamic_slice(x, ((1 - my_x) * (m - 1), 0), (1, n))
    col_edge = lax.dynamic_slice(x, (0, (1 - my_y) * (n - 1)), (m, 1))
    col_packed = col_edge.reshape(m // LANES, LANES)

    up_edges, dn_edges, recv_col_packed = _exchange(row_edge, col_packed, x)
    recv_col = recv_col_packed.reshape(m, 1)

    return _stencil(x, up_edges, dn_edges, recv_col)


# device time: 56277 ns/iter; 1.1967x vs baseline; 1.0237x over previous
import jax
import jax.numpy as jnp
from jax import lax
from jax.experimental import pallas as pl
from jax.experimental.pallas import tpu as pltpu

R = 256
LANES = 128
OUT_DTYPE = jnp.bfloat16


def _exchange(row_edge, col_packed, x):
    m, n = x.shape
    nb = m // R
    pk = m // LANES

    def body(row_ref, col_ref, x_ref, up_ref, dn_ref, ocol_ref,
             rsend, rrecv, lsem):
        my_x = lax.axis_index("x")
        my_y = lax.axis_index("y")

        for i in range(1, nb):
            pltpu.make_async_copy(
                x_ref.at[pl.ds(i * R - 1, 1), :],
                up_ref.at[pl.ds(i, 1), :],
                lsem.at[0, i],
            ).start()
        for i in range(nb - 1):
            pltpu.make_async_copy(
                x_ref.at[pl.ds((i + 1) * R, 1), :],
                dn_ref.at[pl.ds(i, 1), :],
                lsem.at[1, i],
            ).start()

        bsem = pltpu.get_barrier_semaphore()
        for nbr in ((1 - my_x, my_y), (my_x, 1 - my_y)):
            pl.semaphore_signal(
                bsem, inc=1, device_id=nbr,
                device_id_type=pl.DeviceIdType.MESH,
            )
        pl.semaphore_wait(bsem, 2)

        def row_rdma():
            return pltpu.make_async_remote_copy(
                src_ref=row_ref,
                dst_ref=up_ref.at[pl.ds(0, 1), :],
                send_sem=rsend.at[0],
                recv_sem=rrecv.at[0],
                device_id=(1 - my_x, my_y),
                device_id_type=pl.DeviceIdType.MESH,
            )

        def row_rdma_s():
            return pltpu.make_async_remote_copy(
                src_ref=row_ref,
                dst_ref=dn_ref.at[pl.ds(nb - 1, 1), :],
                send_sem=rsend.at[0],
                recv_sem=rrecv.at[0],
                device_id=(1 - my_x, my_y),
                device_id_type=pl.DeviceIdType.MESH,
            )

        @pl.when(my_x == 0)
        def _():
            row_rdma().start()

        @pl.when(my_x == 1)
        def _():
            row_rdma_s().start()

        c_rdma = pltpu.make_async_remote_copy(
            src_ref=col_ref,
            dst_ref=ocol_ref,
            send_sem=rsend.at[1],
            recv_sem=rrecv.at[1],
            device_id=(my_x, 1 - my_y),
            device_id_type=pl.DeviceIdType.MESH,
        )
        c_rdma.start()

        @pl.when(my_x == 0)
        def _():
            row_rdma().wait()

        @pl.when(my_x == 1)
        def _():
            row_rdma_s().wait()

        c_rdma.wait()

        for i in range(1, nb):
            pltpu.make_async_copy(
                x_ref.at[pl.ds(i * R - 1, 1), :],
                up_ref.at[pl.ds(i, 1), :],
                lsem.at[0, i],
            ).wait()
        for i in range(nb - 1):
            pltpu.make_async_copy(
                x_ref.at[pl.ds((i + 1) * R, 1), :],
                dn_ref.at[pl.ds(i, 1), :],
                lsem.at[1, i],
            ).wait()

    return pl.pallas_call(
        body,
        out_shape=[
            jax.ShapeDtypeStruct((nb, n), x.dtype),
            jax.ShapeDtypeStruct((nb, n), x.dtype),
            jax.ShapeDtypeStruct((pk, LANES), x.dtype),
        ],
        in_specs=[
            pl.BlockSpec(memory_space=pltpu.VMEM),
            pl.BlockSpec(memory_space=pltpu.VMEM),
            pl.BlockSpec(memory_space=pltpu.MemorySpace.HBM),
        ],
        out_specs=[
            pl.BlockSpec(memory_space=pltpu.VMEM),
            pl.BlockSpec(memory_space=pltpu.VMEM),
            pl.BlockSpec(memory_space=pltpu.VMEM),
        ],
        scratch_shapes=[
            pltpu.SemaphoreType.DMA((2,)),
            pltpu.SemaphoreType.DMA((2,)),
            pltpu.SemaphoreType.DMA((2, nb)),
        ],
        compiler_params=pltpu.CompilerParams(collective_id=0),
    )(row_edge, col_packed, x)


def _stencil(x, up_edges, dn_edges, recv_col):
    m, n = x.shape
    nb = m // R

    def body(x_ref, up_ref, dn_ref, wcol_ref, ecol_ref, out_ref):
        i = pl.program_id(0)
        my_x = lax.axis_index("x")
        my_y = lax.axis_index("y")

        half = jnp.bfloat16(0.5)
        eighth = jnp.bfloat16(0.125)
        xb = x_ref[...].astype(OUT_DTYPE)
        up_row = up_ref[pl.ds(i, 1), :].astype(OUT_DTYPE)
        dn_row = dn_ref[pl.ds(i, 1), :].astype(OUT_DTYPE)
        up = jnp.concatenate([up_row, xb[:-1, :]], axis=0)
        dn = jnp.concatenate([xb[1:, :], dn_row], axis=0)
        left = pltpu.roll(xb, 1, 1)
        right = pltpu.roll(xb, n - 1, 1)
        out_ref[...] = half * xb + eighth * ((up + dn) + (left + right))

        wcol = wcol_ref[...].astype(OUT_DTYPE)
        ecol = ecol_ref[...].astype(OUT_DTYPE)
        out_ref[:, 0:1] = half * xb[:, 0:1] + eighth * (
            up[:, 0:1] + dn[:, 0:1] + wcol + xb[:, 1:2]
        )
        out_ref[:, n - 1 : n] = half * xb[:, n - 1 : n] + eighth * (
            up[:, n - 1 : n] + dn[:, n - 1 : n] + xb[:, n - 2 : n - 1] + ecol
        )

        @pl.when(my_y == 0)
        def _():
            out_ref[:, 0:1] = xb[:, 0:1]

        @pl.when(my_y == 1)
        def _():
            out_ref[:, n - 1 : n] = xb[:, n - 1 : n]

        @pl.when((my_x == 0) & (i == 0))
        def _():
            out_ref[0:1, :] = xb[0:1, :]

        @pl.when((my_x == 1) & (i == nb - 1))
        def _():
            out_ref[R - 1 : R, :] = xb[R - 1 : R, :]

    return pl.pallas_call(
        body,
        grid=(nb,),
        out_shape=jax.ShapeDtypeStruct((m, n), OUT_DTYPE),
        in_specs=[
            pl.BlockSpec((R, n), lambda i: (i, 0)),
            pl.BlockSpec((nb, n), lambda i: (0, 0)),
            pl.BlockSpec((nb, n), lambda i: (0, 0)),
            pl.BlockSpec((R, 1), lambda i: (i, 0)),
            pl.BlockSpec((R, 1), lambda i: (i, 0)),
        ],
        out_specs=pl.BlockSpec((R, n), lambda i: (i, 0)),
        compiler_params=pltpu.CompilerParams(
            vmem_limit_bytes=100 * 1024 * 1024,
        ),
    )(x, up_edges, dn_edges, recv_col, recv_col)


def kernel(x):
    m, n = x.shape
    my_x = lax.axis_index("x")
    my_y = lax.axis_index("y")

    row_edge = lax.dynamic_slice(x, ((1 - my_x) * (m - 1), 0), (1, n))
    col_edge = lax.dynamic_slice(x, (0, (1 - my_y) * (n - 1)), (m, 1))
    col_packed = col_edge.reshape(m // LANES, LANES)

    up_edges, dn_edges, recv_col_packed = _exchange(row_edge, col_packed, x)
    recv_col = recv_col_packed.reshape(m, 1)

    return _stencil(x, up_edges, dn_edges, recv_col)
